# Initial kernel scaffold; baseline (speedup 1.0000x reference)
import jax
import jax.numpy as jnp
from jax.experimental import pallas as pl


def kernel(x, lut):
    raise NotImplementedError("write your pallas kernel here")


# speedup vs baseline: 1.1983x; 1.1983x over previous
import functools
import math

import jax
import jax.numpy as jnp
from jax import lax
from jax.experimental import pallas as pl
from jax.experimental.pallas import tpu as pltpu
from jax.experimental.pallas import tpu_sc as plsc

D_MODEL = 32
SCALE = math.sqrt(D_MODEL)

NC = 2
NS = 16
NW = NC * NS

CHUNK = 128


def _make_kernel(total_rows):
    chunks_per_w = total_rows // (NW * CHUNK)
    rows_per_w = chunks_per_w * CHUNK

    @functools.partial(
        pl.kernel,
        out_type=jax.ShapeDtypeStruct((total_rows, D_MODEL), jnp.float32),
        mesh=plsc.VectorSubcoreMesh(core_axis_name="c", subcore_axis_name="s"),
        scratch_types=[
            pltpu.VMEM((chunks_per_w, CHUNK), jnp.int32),
            pltpu.VMEM((CHUNK, D_MODEL), jnp.float32),
            pltpu.VMEM((CHUNK, D_MODEL), jnp.float32),
            pltpu.SemaphoreType.DMA,
        ],
        compiler_params=pltpu.CompilerParams(use_tc_tiling_on_sc=False),
    )
    def body(idx_hbm, table_hbm, out_hbm, idx_v, gbuf, obuf, gsem):
        c = lax.axis_index("c")
        s = lax.axis_index("s")
        wid = s * NC + c
        base = wid * rows_per_w
        pltpu.sync_copy(idx_hbm.at[wid], idx_v)

        def chunk_body(g, carry):
            pltpu.async_copy(table_hbm.at[idx_v.at[g]], gbuf, gsem).wait()

            def row_body(r, rc):
                obuf[r, 0:16] = gbuf[r, 0:16] * SCALE
                obuf[r, 16:32] = gbuf[r, 16:32] * SCALE
                return rc

            lax.fori_loop(0, CHUNK, row_body, 0)
            pltpu.sync_copy(obuf, out_hbm.at[pl.ds(base + g * CHUNK, CHUNK)])
            return carry

        lax.fori_loop(0, chunks_per_w, chunk_body, 0)

    return body


def kernel(x, lut):
    total = x.shape[0] * x.shape[1]
    chunks_per_w = total // (NW * CHUNK)
    xi = jnp.asarray(x, jnp.int32).reshape(NW, chunks_per_w, CHUNK)
    out = _make_kernel(total)(xi, lut)
    return out.reshape(x.shape[0], x.shape[1], D_MODEL)


# speedup vs baseline: 1.3872x; 1.1576x over previous
import functools
import math

import jax
import jax.numpy as jnp
from jax import lax
from jax.experimental import pallas as pl
from jax.experimental.pallas import tpu as pltpu
from jax.experimental.pallas import tpu_sc as plsc

D_MODEL = 32
SCALE = math.sqrt(D_MODEL)

NC = 2
NS = 16
NW = NC * NS

CHUNK = 128
NBUF = 8


def _make_kernel(total_rows):
    chunks_per_w = total_rows // (NW * CHUNK)
    rows_per_w = chunks_per_w * CHUNK
    niter = chunks_per_w // NBUF

    @functools.partial(
        pl.kernel,
        out_type=jax.ShapeDtypeStruct((total_rows, D_MODEL), jnp.float32),
        mesh=plsc.VectorSubcoreMesh(core_axis_name="c", subcore_axis_name="s"),
        scratch_types=[
            pltpu.VMEM((chunks_per_w, CHUNK), jnp.int32),
            pltpu.VMEM((NBUF, CHUNK, D_MODEL), jnp.float32),
            pltpu.VMEM((NBUF, CHUNK, D_MODEL), jnp.float32),
        ]
        + [pltpu.SemaphoreType.DMA] * (2 * NBUF),
        compiler_params=pltpu.CompilerParams(use_tc_tiling_on_sc=False),
    )
    def body(idx_hbm, table_hbm, out_hbm, idx_v, gbuf, obuf, *sems):
        gsems = sems[:NBUF]
        osems = sems[NBUF:]
        c = lax.axis_index("c")
        s = lax.axis_index("s")
        wid = s * NC + c
        base = wid * rows_per_w
        pltpu.sync_copy(idx_hbm.at[wid], idx_v)

        def issue_gather(g, b):
            pltpu.async_copy(table_hbm.at[idx_v.at[g]], gbuf.at[b], gsems[b])

        def wait_gather(g, b):
            pltpu.make_async_copy(
                table_hbm.at[idx_v.at[g]], gbuf.at[b], gsems[b]
            ).wait()

        def issue_out(g, b):
            pltpu.async_copy(
                obuf.at[b], out_hbm.at[pl.ds(base + g * CHUNK, CHUNK)], osems[b]
            )

        def wait_out(g, b):
            pltpu.make_async_copy(
                obuf.at[b], out_hbm.at[pl.ds(base + g * CHUNK, CHUNK)], osems[b]
            ).wait()

        def scale(b):
            def row_body(r, rc):
                obuf[b, r, 0:16] = gbuf[b, r, 0:16] * SCALE
                obuf[b, r, 16:32] = gbuf[b, r, 16:32] * SCALE
                return rc

            lax.fori_loop(0, CHUNK, row_body, 0, unroll=8)

        for b in range(NBUF):
            issue_gather(b, b)

        for b in range(NBUF):
            wait_gather(b, b)
            scale(b)
            issue_gather(b + NBUF, b)
            issue_out(b, b)

        def mid(i, carry):
            for b in range(NBUF):
                g = i * NBUF + b
                wait_gather(g, b)
                wait_out(g - NBUF, b)
                scale(b)
                issue_gather(g + NBUF, b)
                issue_out(g, b)
            return carry

        lax.fori_loop(1, niter - 1, mid, 0)

        for b in range(NBUF):
            g = (niter - 1) * NBUF + b
            wait_gather(g, b)
            wait_out(g - NBUF, b)
            scale(b)
            issue_out(g, b)
        for b in range(NBUF):
            wait_out((niter - 1) * NBUF + b, b)

    return body


def kernel(x, lut):
    total = x.shape[0] * x.shape[1]
    chunks_per_w = total // (NW * CHUNK)
    xi = jnp.asarray(x, jnp.int32).reshape(NW, chunks_per_w, CHUNK)
    out = _make_kernel(total)(xi, lut)
    return out.reshape(x.shape[0], x.shape[1], D_MODEL)
